# Initial kernel scaffold; baseline (speedup 1.0000x reference)
import jax
import jax.numpy as jnp
from jax.experimental import pallas as pl


def kernel(x, subject_labels, W1, b1, W2, b2):
    raise NotImplementedError("write your pallas kernel here")


# speedup vs baseline: 10.4972x; 10.4972x over previous
import functools

import jax
import jax.numpy as jnp
from jax import lax
from jax.experimental import pallas as pl
from jax.experimental.pallas import tpu as pltpu
from jax.experimental.pallas import tpu_sc as plsc

LANES = 16
CHUNK = 256


def _make_seg_mean(n_rows: int, d: int, nseg: int):
  nlane_blocks = d // LANES
  mesh = plsc.VectorSubcoreMesh(core_axis_name="c", subcore_axis_name="s")

  @functools.partial(
      pl.kernel,
      out_type=jax.ShapeDtypeStruct((nseg, d), jnp.float32),
      mesh=mesh,
      scratch_types=[
          pltpu.VMEM((CHUNK, d), jnp.float32),
          pltpu.VMEM((48,), jnp.int32),
          pltpu.VMEM((d,), jnp.float32),
      ],
  )
  def seg_mean(x_hbm, bounds_hbm, out_hbm, buf, bnd, row):
    num_cores = jax.lax.axis_size("c")
    sid = lax.axis_index("s") * num_cores + lax.axis_index("c")
    pltpu.sync_copy(bounds_hbm, bnd)
    bv = bnd[pl.ds(sid, LANES)]
    r0 = bv[0]
    r1 = bv[1]
    base0 = (r0 // 8) * 8
    nch = lax.div(r1 - base0 + (CHUNK - 1), CHUNK)

    def row_body(r, accs):
      return tuple(accs[k] + buf[r, pl.ds(LANES * k, LANES)]
                   for k in range(nlane_blocks))

    def chunk_body(j, accs):
      w0 = jnp.minimum(base0 + j * CHUNK, n_rows - CHUNK)
      pltpu.sync_copy(x_hbm.at[pl.ds(w0, CHUNK)], buf)
      a = jnp.maximum(r0, w0) - w0
      b = jnp.minimum(r1, w0 + CHUNK) - w0
      return lax.fori_loop(a, b, row_body, accs)

    zero = jnp.zeros((LANES,), jnp.float32)
    accs = lax.fori_loop(0, nch, chunk_body,
                         tuple(zero for _ in range(nlane_blocks)))

    cnt = jnp.full((LANES,), r1 - r0, jnp.int32).astype(jnp.float32)
    inv = 1.0 / cnt
    for k in range(nlane_blocks):
      row[pl.ds(LANES * k, LANES)] = accs[k] * inv
    pltpu.sync_copy(row, out_hbm.at[sid])

  return seg_mean


def _mlp_body(feats_ref, w1_ref, b1_ref, w2_ref, b2_ref, out_ref):
  h = jnp.dot(feats_ref[...], w1_ref[...],
              preferred_element_type=jnp.float32) + b1_ref[...]
  h = jnp.maximum(h, 0.0)
  logits = jnp.dot(h, w2_ref[...],
                   preferred_element_type=jnp.float32) + b2_ref[...]
  m = jnp.max(logits, axis=-1, keepdims=True)
  e = jnp.exp(logits - m)
  out_ref[...] = e / jnp.sum(e, axis=-1, keepdims=True)


def kernel(x, subject_labels, W1, b1, W2, b2):
  n, d = x.shape
  nseg = b2.shape[0]
  labels = subject_labels.astype(jnp.int32)

  bounds = jnp.searchsorted(
      labels, jnp.arange(nseg + 1, dtype=jnp.int32), side="left"
  ).astype(jnp.int32)
  bounds_p = jnp.zeros((48,), jnp.int32).at[: nseg + 1].set(bounds)

  feats = _make_seg_mean(n, d, nseg)(x, bounds_p)

  probs = pl.pallas_call(
      _mlp_body,
      out_shape=jax.ShapeDtypeStruct((nseg, nseg), jnp.float32),
  )(feats, W1, b1.reshape(1, -1), W2, b2.reshape(1, -1))

  counts = bounds[1:] - bounds[:-1]
  vals = jnp.arange(nseg, dtype=jnp.int32)
  present = counts > 0
  order = jnp.argsort(jnp.where(present, vals, jnp.int32(nseg)))
  compact = vals[order]
  k = jnp.sum(present.astype(jnp.int32))
  uids = jnp.where(vals < k, compact, compact[0]).astype(subject_labels.dtype)

  return (probs, uids)


# speedup vs baseline: 14.8701x; 1.4166x over previous
import functools

import jax
import jax.numpy as jnp
from jax import lax
from jax.experimental import pallas as pl
from jax.experimental.pallas import tpu as pltpu
from jax.experimental.pallas import tpu_sc as plsc

LANES = 16
CHUNK = 256


def _make_seg_mean(n_rows: int, d: int, nseg: int):
  nlane_blocks = d // LANES
  mesh = plsc.VectorSubcoreMesh(core_axis_name="c", subcore_axis_name="s")

  @functools.partial(
      pl.kernel,
      out_type=jax.ShapeDtypeStruct((nseg, d), jnp.float32),
      mesh=mesh,
      scratch_types=[
          pltpu.VMEM((CHUNK, d), jnp.float32),
          pltpu.VMEM((CHUNK, d), jnp.float32),
          pltpu.VMEM((48,), jnp.int32),
          pltpu.VMEM((d,), jnp.float32),
          pltpu.SemaphoreType.DMA,
          pltpu.SemaphoreType.DMA,
      ],
  )
  def seg_mean(x_hbm, bounds_hbm, out_hbm, buf0, buf1, bnd, row,
               sem0, sem1):
    num_cores = jax.lax.axis_size("c")
    sid = lax.axis_index("s") * num_cores + lax.axis_index("c")
    pltpu.sync_copy(bounds_hbm, bnd)
    bv = bnd[pl.ds(sid, LANES)]
    r0 = bv[0]
    r1 = bv[1]
    base0 = (r0 // 8) * 8
    nch = lax.div(r1 - base0 + (CHUNK - 1), CHUNK)

    def win_start(j):
      return jnp.minimum(base0 + j * CHUNK, n_rows - CHUNK)

    def dma_start(j, buf, sem):
      pltpu.make_async_copy(x_hbm.at[pl.ds(win_start(j), CHUNK)],
                            buf, sem).start()

    def process(j, buf, sem, pfbuf, pfsem, accs):
      @pl.when(j + 1 < nch)
      def _():
        dma_start(j + 1, pfbuf, pfsem)

      @pl.when(j < nch)
      def _():
        pltpu.make_async_copy(x_hbm.at[pl.ds(0, CHUNK)], buf, sem).wait()

      w0 = win_start(j)
      a = jnp.maximum(r0, w0) - w0
      b = jnp.where(j < nch, jnp.minimum(r1, w0 + CHUNK) - w0, a)

      def row_body(r, accs):
        return tuple(accs[k] + buf[r, pl.ds(LANES * k, LANES)]
                     for k in range(nlane_blocks))

      return lax.fori_loop(a, b, row_body, accs)

    @pl.when(nch > 0)
    def _():
      dma_start(0, buf0, sem0)

    def pair_body(p, accs):
      j0 = 2 * p
      accs = process(j0, buf0, sem0, buf1, sem1, accs)
      accs = process(j0 + 1, buf1, sem1, buf0, sem0, accs)
      return accs

    zero = jnp.zeros((LANES,), jnp.float32)
    accs = lax.fori_loop(0, (nch + 1) // 2, pair_body,
                         tuple(zero for _ in range(nlane_blocks)))

    cnt = jnp.full((LANES,), r1 - r0, jnp.int32).astype(jnp.float32)
    inv = 1.0 / cnt
    for k in range(nlane_blocks):
      row[pl.ds(LANES * k, LANES)] = accs[k] * inv
    pltpu.sync_copy(row, out_hbm.at[sid])

  return seg_mean


def _mlp_body(feats_ref, w1_ref, b1_ref, w2_ref, b2_ref, out_ref):
  h = jnp.dot(feats_ref[...], w1_ref[...],
              preferred_element_type=jnp.float32) + b1_ref[...]
  h = jnp.maximum(h, 0.0)
  logits = jnp.dot(h, w2_ref[...],
                   preferred_element_type=jnp.float32) + b2_ref[...]
  m = jnp.max(logits, axis=-1, keepdims=True)
  e = jnp.exp(logits - m)
  out_ref[...] = e / jnp.sum(e, axis=-1, keepdims=True)


def kernel(x, subject_labels, W1, b1, W2, b2):
  n, d = x.shape
  nseg = b2.shape[0]
  labels = subject_labels.astype(jnp.int32)

  bounds = jnp.searchsorted(
      labels, jnp.arange(nseg + 1, dtype=jnp.int32), side="left"
  ).astype(jnp.int32)
  bounds_p = jnp.zeros((48,), jnp.int32).at[: nseg + 1].set(bounds)

  feats = _make_seg_mean(n, d, nseg)(x, bounds_p)

  probs = pl.pallas_call(
      _mlp_body,
      out_shape=jax.ShapeDtypeStruct((nseg, nseg), jnp.float32),
  )(feats, W1, b1.reshape(1, -1), W2, b2.reshape(1, -1))

  counts = bounds[1:] - bounds[:-1]
  vals = jnp.arange(nseg, dtype=jnp.int32)
  present = counts > 0
  order = jnp.argsort(jnp.where(present, vals, jnp.int32(nseg)))
  compact = vals[order]
  k = jnp.sum(present.astype(jnp.int32))
  uids = jnp.where(vals < k, compact, compact[0]).astype(subject_labels.dtype)

  return (probs, uids)


# speedup vs baseline: 20.8866x; 1.4046x over previous
import functools

import jax
import jax.numpy as jnp
from jax import lax
from jax.experimental import pallas as pl
from jax.experimental.pallas import tpu as pltpu
from jax.experimental.pallas import tpu_sc as plsc

LANES = 16
CHUNK = 256
NSUB = 16


def _make_seg_mean(n_rows: int, d: int, nseg: int):
  nlane_blocks = d // LANES
  pt = n_rows // NSUB
  bits = []
  b = 1
  while b <= pt:
    b *= 2
  while b >= 1:
    bits.append(b)
    b //= 2
  mesh = plsc.VectorSubcoreMesh(core_axis_name="c", subcore_axis_name="s")

  @functools.partial(
      pl.kernel,
      out_type=(
          jax.ShapeDtypeStruct((nseg, d), jnp.float32),
          jax.ShapeDtypeStruct((48,), jnp.int32),
          jax.ShapeDtypeStruct((2 * NSUB, 2 * LANES), jnp.int32),
      ),
      mesh=mesh,
      compiler_params=pltpu.CompilerParams(needs_layout_passes=False),
      scratch_types=[
          pltpu.VMEM((pt,), jnp.int32),
          pltpu.VMEM((CHUNK, d), jnp.float32),
          pltpu.VMEM((CHUNK, d), jnp.float32),
          pltpu.VMEM((48,), jnp.int32),
          pltpu.VMEM((2 * LANES,), jnp.int32),
          pltpu.VMEM((NSUB, 2 * LANES), jnp.int32),
          pltpu.VMEM((d,), jnp.float32),
          pltpu.SemaphoreType.DMA,
          pltpu.SemaphoreType.DMA,
      ],
  )
  def seg_mean(x_hbm, labels_hbm, out_hbm, bounds_hbm, stage_hbm,
               lab, buf0, buf1, bnd, cbuf, call, row, sem0, sem1):
    num_cores = jax.lax.axis_size("c")
    cid = lax.axis_index("c")
    tid = lax.axis_index("s")
    sid = tid * num_cores + cid

    pltpu.sync_copy(labels_hbm.at[pl.ds(tid * pt, pt)], lab)
    iota = lax.iota(jnp.int32, LANES)
    q0 = iota + 1
    q1 = iota + 17
    pos0 = jnp.zeros((LANES,), jnp.int32)
    pos1 = jnp.zeros((LANES,), jnp.int32)
    for bit in bits:
      cand0 = pos0 + bit
      cand1 = pos1 + bit
      v0 = plsc.load_gather(lab, [jnp.minimum(cand0, pt) - 1])
      v1 = plsc.load_gather(lab, [jnp.minimum(cand1, pt) - 1])
      pos0 = jnp.where((cand0 <= pt) & (v0 < q0), cand0, pos0)
      pos1 = jnp.where((cand1 <= pt) & (v1 < q1), cand1, pos1)
    cbuf[pl.ds(0, LANES)] = pos0
    cbuf[pl.ds(LANES, LANES)] = pos1
    pltpu.sync_copy(cbuf, stage_hbm.at[cid * NSUB + tid])
    plsc.subcore_barrier()
    pltpu.sync_copy(stage_hbm.at[pl.ds(cid * NSUB, NSUB)], call)
    g0 = jnp.zeros((LANES,), jnp.int32)
    g1 = jnp.zeros((LANES,), jnp.int32)
    for i in range(NSUB):
      g0 = g0 + call[i, pl.ds(0, LANES)]
      g1 = g1 + call[i, pl.ds(LANES, LANES)]
    bnd[pl.ds(0, LANES)] = g0
    bnd[pl.ds(LANES, LANES)] = g1

    @pl.when(sid == 0)
    def _():
      pltpu.sync_copy(bnd, bounds_hbm)

    off = jnp.maximum(sid - 1, 0)
    bv = bnd[pl.ds(off, LANES)]
    is0 = sid == 0
    r0 = jnp.where(is0, 0, bv[0])
    r1 = jnp.where(is0, bv[0], bv[1])
    base0 = (r0 // 8) * 8
    nch = lax.div(r1 - base0 + (CHUNK - 1), CHUNK)

    def win_start(j):
      return jnp.minimum(base0 + j * CHUNK, n_rows - CHUNK)

    def dma_start(j, buf, sem):
      pltpu.make_async_copy(x_hbm.at[pl.ds(win_start(j), CHUNK)],
                            buf, sem).start()

    def process(j, buf, sem, pfbuf, pfsem, accs):
      @pl.when(j + 1 < nch)
      def _():
        dma_start(j + 1, pfbuf, pfsem)

      @pl.when(j < nch)
      def _():
        pltpu.make_async_copy(x_hbm.at[pl.ds(0, CHUNK)], buf, sem).wait()

      w0 = win_start(j)
      a = jnp.maximum(r0, w0) - w0
      b = jnp.where(j < nch, jnp.minimum(r1, w0 + CHUNK) - w0, a)

      def row_body(r, accs):
        return tuple(accs[k] + buf[r, pl.ds(LANES * k, LANES)]
                     for k in range(nlane_blocks))

      return lax.fori_loop(a, b, row_body, accs)

    @pl.when(nch > 0)
    def _():
      dma_start(0, buf0, sem0)

    def pair_body(p, accs):
      j0 = 2 * p
      accs = process(j0, buf0, sem0, buf1, sem1, accs)
      accs = process(j0 + 1, buf1, sem1, buf0, sem0, accs)
      return accs

    zero = jnp.zeros((LANES,), jnp.float32)
    accs = lax.fori_loop(0, (nch + 1) // 2, pair_body,
                         tuple(zero for _ in range(nlane_blocks)))

    cnt = jnp.full((LANES,), r1 - r0, jnp.int32).astype(jnp.float32)
    inv = 1.0 / cnt
    for k in range(nlane_blocks):
      row[pl.ds(LANES * k, LANES)] = accs[k] * inv
    pltpu.sync_copy(row, out_hbm.at[sid])

  return seg_mean


def _mlp_body(feats_ref, w1_ref, b1_ref, w2_ref, b2_ref, out_ref):
  h = jnp.dot(feats_ref[...], w1_ref[...],
              preferred_element_type=jnp.float32) + b1_ref[...]
  h = jnp.maximum(h, 0.0)
  logits = jnp.dot(h, w2_ref[...],
                   preferred_element_type=jnp.float32) + b2_ref[...]
  m = jnp.max(logits, axis=-1, keepdims=True)
  e = jnp.exp(logits - m)
  out_ref[...] = e / jnp.sum(e, axis=-1, keepdims=True)


def kernel(x, subject_labels, W1, b1, W2, b2):
  n, d = x.shape
  nseg = b2.shape[0]
  labels = subject_labels.astype(jnp.int32)

  feats, bnd_out, _stage = _make_seg_mean(n, d, nseg)(x, labels)

  probs = pl.pallas_call(
      _mlp_body,
      out_shape=jax.ShapeDtypeStruct((nseg, nseg), jnp.float32),
  )(feats, W1, b1.reshape(1, -1), W2, b2.reshape(1, -1))

  bounds = jnp.concatenate([jnp.zeros((1,), jnp.int32), bnd_out[:nseg]])
  counts = bounds[1:] - bounds[:-1]
  vals = jnp.arange(nseg, dtype=jnp.int32)
  present = counts > 0
  order = jnp.argsort(jnp.where(present, vals, jnp.int32(nseg)))
  compact = vals[order]
  k = jnp.sum(present.astype(jnp.int32))
  uids = jnp.where(vals < k, compact, compact[0]).astype(subject_labels.dtype)

  return (probs, uids)
